# baseline (speedup 1.0000x reference)
import jax
import jax.numpy as jnp
from jax.experimental import pallas as pl

NUM_NODES = 100000
DIM = 64
MOMENTUM = 0.9


def kernel(x, idx, hidden, variance, W_ih, W_hh, b_ih, b_hh):
    B = x.shape[0]
    pos = jnp.arange(B, dtype=jnp.int32)
    last = jnp.full((NUM_NODES,), -1, jnp.int32).at[idx].max(pos)

    h_old = hidden[idx]
    gi = x @ W_ih.T + b_ih
    gh = h_old @ W_hh.T + b_hh
    i_r, i_z, i_n = jnp.split(gi, 3, axis=1)
    h_r, h_z, h_n = jnp.split(gh, 3, axis=1)
    r = jax.nn.sigmoid(i_r + h_r)
    z = jax.nn.sigmoid(i_z + h_z)
    n = jnp.tanh(i_n + r * h_n)
    h_new = (1.0 - z) * n + z * h_old
    delta = h_new - h_old
    var_rows = MOMENTUM * variance[idx] + (1.0 - MOMENTUM) * jnp.square(delta)

    upd = last >= 0
    win = jnp.maximum(last, 0)
    out0 = jnp.where(upd[:, None], h_new[win], hidden)
    out1 = jnp.where(upd[:, None], var_rows[win], variance)
    return jnp.stack([out0, out1], axis=0)


# speedup vs baseline: 2.1111x; 2.1111x over previous
import functools

import jax
import jax.numpy as jnp
from jax import lax
from jax.experimental import pallas as pl
from jax.experimental.pallas import tpu as pltpu
from jax.experimental.pallas import tpu_sc as plsc

NUM_NODES = 100000
DIM = 64
MOMENTUM = 0.9
_NC, _NS, _L = 2, 16, 16
_NW = _NC * _NS


def _sc_gather(hidden, variance, idx):
    B = idx.shape[0]
    bpw = B // _NW
    mesh = plsc.VectorSubcoreMesh(core_axis_name="c", subcore_axis_name="s")

    @functools.partial(
        pl.kernel,
        mesh=mesh,
        out_type=pltpu.HBM((B, 2 * DIM), jnp.float32),
        scratch_types=[
            pltpu.VMEM((bpw,), jnp.int32),
            pltpu.VMEM((bpw, 2 * DIM), jnp.float32),
            pltpu.SemaphoreType.DMA,
        ],
    )
    def k(hid_hbm, var_hbm, idx_hbm, hv_hbm, idx_v, rows, s1):
        wid = lax.axis_index("s") * _NC + lax.axis_index("c")
        base = wid * bpw
        pltpu.sync_copy(idx_hbm.at[pl.ds(base, bpw)], idx_v)

        def grp(g, _):
            v = idx_v[pl.ds(g * _L, _L)]
            for j in range(_L):
                n = v[j]
                i = g * _L + j
                pltpu.async_copy(hid_hbm.at[n], rows.at[i, pl.ds(0, DIM)], s1)
                pltpu.async_copy(var_hbm.at[n], rows.at[i, pl.ds(DIM, DIM)], s1)
            return _

        lax.fori_loop(0, bpw // _L, grp, 0)
        pltpu.make_async_copy(hv_hbm.at[pl.ds(base, bpw)], rows, s1).wait()
        pltpu.sync_copy(rows, hv_hbm.at[pl.ds(base, bpw)])

    return k(hidden, variance, idx)


def _tc_gru(x, hv, wih_t, whh_t, b_r, b_z, b_in, b_hn):
    B = x.shape[0]
    blk = 2048

    def body(x_ref, hv_ref, wi_ref, wh_ref, br_ref, bz_ref, bi_ref, bh_ref,
             hn_ref, vn_ref):
        xb = x_ref[...]
        hb = hv_ref[:, 0:DIM]
        vb = hv_ref[:, DIM:2 * DIM]
        gi = jnp.dot(xb, wi_ref[...], preferred_element_type=jnp.float32)
        gh = jnp.dot(hb, wh_ref[...], preferred_element_type=jnp.float32)
        r = jax.nn.sigmoid(gi[:, 0:DIM] + gh[:, 0:DIM] + br_ref[...])
        z = jax.nn.sigmoid(gi[:, DIM:2 * DIM] + gh[:, DIM:2 * DIM] + bz_ref[...])
        n = jnp.tanh(gi[:, 2 * DIM:] + bi_ref[...] + r * (gh[:, 2 * DIM:] + bh_ref[...]))
        hn = (1.0 - z) * n + z * hb
        d = hn - hb
        hn_ref[...] = hn
        vn_ref[...] = MOMENTUM * vb + (1.0 - MOMENTUM) * d * d

    row_spec = pl.BlockSpec((blk, DIM), lambda i: (i, 0))
    wide_spec = pl.BlockSpec((blk, 2 * DIM), lambda i: (i, 0))
    full = pl.BlockSpec((DIM, 3 * DIM), lambda i: (0, 0))
    bias = pl.BlockSpec((1, DIM), lambda i: (0, 0))
    return pl.pallas_call(
        body,
        grid=(B // blk,),
        in_specs=[row_spec, wide_spec, full, full, bias, bias, bias, bias],
        out_specs=[row_spec, row_spec],
        out_shape=(
            jax.ShapeDtypeStruct((B, DIM), jnp.float32),
            jax.ShapeDtypeStruct((B, DIM), jnp.float32),
        ),
    )(x, hv, wih_t, whh_t, b_r, b_z, b_in, b_hn)


def kernel(x, idx, hidden, variance, W_ih, W_hh, b_ih, b_hh):
    idx = idx.astype(jnp.int32)
    hv = _sc_gather(hidden, variance, idx)
    wih_t = W_ih.T
    whh_t = W_hh.T
    b_r = (b_ih[0:DIM] + b_hh[0:DIM]).reshape(1, DIM)
    b_z = (b_ih[DIM:2 * DIM] + b_hh[DIM:2 * DIM]).reshape(1, DIM)
    b_in = b_ih[2 * DIM:].reshape(1, DIM)
    b_hn = b_hh[2 * DIM:].reshape(1, DIM)
    h_new, var_rows = _tc_gru(x, hv, wih_t, whh_t, b_r, b_z, b_in, b_hn)
    hidden_new = hidden.at[idx].set(h_new)
    variance_new = variance.at[idx].set(var_rows)
    return jnp.stack([hidden_new, variance_new], axis=0)
